# Initial kernel scaffold; baseline (speedup 1.0000x reference)
import jax
import jax.numpy as jnp
from jax.experimental import pallas as pl


def kernel(x, edge_index, W_lin, b_lin, W1_1, W1_2):
    raise NotImplementedError("write your pallas kernel here")


# speedup vs baseline: 2.4838x; 2.4838x over previous
import functools
import math

import jax
import jax.numpy as jnp
from jax import lax
from jax.experimental import pallas as pl
from jax.experimental.pallas import tpu as pltpu
from jax.experimental.pallas import tpu_sc as plsc

N_NODES_K = 10000
N_EDGES_K = 320000
HID = 128
ALPHA_K = 0.1
THETA_K = 0.5

NC = 2
NS = 16
NW = NC * NS

CHUNK = 128
EDGES_PER_TILE = 10240
NCHUNK = EDGES_PER_TILE // CHUNK
E_PAD = NW * EDGES_PER_TILE

N_PAD = 10112
TILE_ROWS = N_PAD // NS


def _sc_segment_sum(src_p, dst_p, h):
    mesh = plsc.VectorSubcoreMesh(core_axis_name="c", subcore_axis_name="s")

    @functools.partial(
        pl.kernel,
        out_type=jax.ShapeDtypeStruct((NC, N_PAD, HID), jnp.float32),
        mesh=mesh,
        scratch_types=[
            pltpu.VMEM_SHARED((N_PAD, HID), jnp.float32),
            pltpu.VMEM((CHUNK,), jnp.int32),
            pltpu.VMEM((CHUNK,), jnp.int32),
            pltpu.VMEM((CHUNK, HID), jnp.float32),
            pltpu.SemaphoreType.DMA,
        ],
    )
    def scatter_kernel(src_hbm, dst_hbm, h_hbm, out_hbm, acc, si, di, rb, sem):
        cid = lax.axis_index("c")
        sid = lax.axis_index("s")
        row0 = pl.multiple_of(sid * TILE_ROWS, 8)

        zero16 = jnp.zeros((16,), jnp.float32)

        def zbody(r, carry):
            for c in range(HID // 16):
                rb[r, pl.ds(c * 16, 16)] = zero16
            return carry

        lax.fori_loop(0, CHUNK, zbody, 0)
        for j in range(TILE_ROWS // CHUNK):
            pltpu.sync_copy(rb, acc.at[pl.ds(row0 + j * CHUNK, CHUNK)])
        rem = TILE_ROWS % CHUNK
        if rem:
            pltpu.sync_copy(
                rb.at[pl.ds(0, rem)],
                acc.at[pl.ds(row0 + (TILE_ROWS // CHUNK) * CHUNK, rem)],
            )
        plsc.subcore_barrier()

        ebase = (cid * NS + sid) * EDGES_PER_TILE

        def body(ci, carry):
            base = pl.multiple_of(ebase + ci * CHUNK, CHUNK)
            pltpu.sync_copy(src_hbm.at[pl.ds(base, CHUNK)], si)
            pltpu.sync_copy(dst_hbm.at[pl.ds(base, CHUNK)], di)
            pltpu.async_copy(h_hbm.at[si], rb, sem).wait()
            pltpu.sync_copy(rb, acc.at[di], add=True)
            return carry

        lax.fori_loop(0, NCHUNK, body, 0)
        plsc.subcore_barrier()

        pltpu.sync_copy(
            acc.at[pl.ds(row0, TILE_ROWS)],
            out_hbm.at[cid, pl.ds(row0, TILE_ROWS)],
        )

    return scatter_kernel(src_p, dst_p, h)


def _lin_relu(x, W, b):
    def body(x_ref, w_ref, b_ref, o_ref):
        acc = jnp.dot(x_ref[...], w_ref[...], preferred_element_type=jnp.float32)
        o_ref[...] = jnp.maximum(acc + b_ref[...], 0.0)

    return pl.pallas_call(
        body,
        grid=(NS,),
        in_specs=[
            pl.BlockSpec((TILE_ROWS, HID), lambda i: (i, 0)),
            pl.BlockSpec((HID, HID), lambda i: (0, 0)),
            pl.BlockSpec((1, HID), lambda i: (0, 0)),
        ],
        out_specs=pl.BlockSpec((TILE_ROWS, HID), lambda i: (i, 0)),
        out_shape=jax.ShapeDtypeStruct((N_PAD, HID), jnp.float32),
    )(x, W, b.reshape(1, HID))


def _layer_update(p0, p1, x0, Wc, beta):
    def body(p0_ref, p1_ref, x0_ref, w_ref, o_ref):
        xx = (p0_ref[...] + p1_ref[...]) * (1.0 - ALPHA_K) + ALPHA_K * x0_ref[...]
        mm = jnp.dot(xx, w_ref[...], preferred_element_type=jnp.float32)
        o_ref[...] = jnp.maximum((1.0 - beta) * xx + beta * mm, 0.0)

    return pl.pallas_call(
        body,
        grid=(NS,),
        in_specs=[
            pl.BlockSpec((TILE_ROWS, HID), lambda i: (i, 0)),
            pl.BlockSpec((TILE_ROWS, HID), lambda i: (i, 0)),
            pl.BlockSpec((TILE_ROWS, HID), lambda i: (i, 0)),
            pl.BlockSpec((HID, HID), lambda i: (0, 0)),
        ],
        out_specs=pl.BlockSpec((TILE_ROWS, HID), lambda i: (i, 0)),
        out_shape=jax.ShapeDtypeStruct((N_PAD, HID), jnp.float32),
    )(p0, p1, x0, Wc)


def kernel(x, edge_index, W_lin, b_lin, W1_1, W1_2):
    src = edge_index[0].astype(jnp.int32)
    dst = edge_index[1].astype(jnp.int32)
    pad = E_PAD - N_EDGES_K
    src_p = jnp.concatenate([src, jnp.zeros((pad,), jnp.int32)])
    dst_p = jnp.concatenate([dst, jnp.full((pad,), N_NODES_K, jnp.int32)])
    x_pad = jnp.pad(x, ((0, N_PAD - N_NODES_K), (0, 0)))

    h = _lin_relu(x_pad, W_lin, b_lin)
    x_cur = h
    for layer, Wc in enumerate((W1_1, W1_2)):
        beta = math.log(THETA_K / (layer + 1) + 1.0)
        p = _sc_segment_sum(src_p, dst_p, x_cur)
        x_cur = _layer_update(p[0], p[1], h, Wc, beta)
    return x_cur[:N_NODES_K]


# speedup vs baseline: 3.4091x; 1.3726x over previous
import functools
import math

import jax
import jax.numpy as jnp
from jax import lax
from jax.experimental import pallas as pl
from jax.experimental.pallas import tpu as pltpu
from jax.experimental.pallas import tpu_sc as plsc

N_NODES_K = 10000
N_EDGES_K = 320000
HID = 128
ALPHA_K = 0.1
THETA_K = 0.5

NC = 2
NS = 16
NW = NC * NS

CHUNK = 80
EDGES_PER_TILE = 10240
NCHUNK = EDGES_PER_TILE // CHUNK
E_PAD = NW * EDGES_PER_TILE

N_PAD = 10112
TILE_ROWS = N_PAD // NS


NBUF = 4


def _sc_segment_sum(src3, dst3, h):
    mesh = plsc.VectorSubcoreMesh(core_axis_name="c", subcore_axis_name="s")

    @functools.partial(
        pl.kernel,
        out_type=jax.ShapeDtypeStruct((NC, N_PAD, HID), jnp.float32),
        mesh=mesh,
        scratch_types=[
            pltpu.VMEM_SHARED((N_PAD, HID), jnp.float32),
            [pltpu.VMEM((CHUNK,), jnp.int32)] * NBUF,
            [pltpu.VMEM((CHUNK,), jnp.int32)] * NBUF,
            [pltpu.VMEM((CHUNK, HID), jnp.float32)] * NBUF,
            [pltpu.SemaphoreType.DMA] * NBUF,
            [pltpu.SemaphoreType.DMA] * NBUF,
            [pltpu.SemaphoreType.DMA] * NBUF,
        ],
    )
    def scatter_kernel(src_hbm, dst_hbm, h_hbm, out_hbm, acc, si, di, rb,
                       gsem, ssem, dsem):
        cid = lax.axis_index("c")
        sid = lax.axis_index("s")
        wid = cid * NS + sid
        row0 = pl.multiple_of(sid * TILE_ROWS, 8)

        zero16 = jnp.zeros((16,), jnp.float32)

        def zbody(r, carry):
            for c in range(HID // 16):
                rb[0][r, pl.ds(c * 16, 16)] = zero16
            return carry

        lax.fori_loop(0, CHUNK, zbody, 0)
        for j in range(TILE_ROWS // CHUNK):
            pltpu.sync_copy(rb[0], acc.at[pl.ds(row0 + j * CHUNK, CHUNK)])
        rem = TILE_ROWS % CHUNK
        if rem:
            pltpu.sync_copy(
                rb[0].at[pl.ds(0, rem)],
                acc.at[pl.ds(row0 + (TILE_ROWS // CHUNK) * CHUNK, rem)],
            )
        plsc.subcore_barrier()

        def issue_idx(ci, b):
            pltpu.async_copy(src_hbm.at[wid, ci], si[b], ssem[b])
            pltpu.async_copy(dst_hbm.at[wid, ci], di[b], dsem[b])

        def issue_gather(ci, b):
            pltpu.make_async_copy(src_hbm.at[wid, ci], si[b], ssem[b]).wait()
            pltpu.async_copy(h_hbm.at[si[b]], rb[b], gsem[b])

        def drain_scatter(ci, b):
            pltpu.make_async_copy(h_hbm.at[si[b]], rb[b], gsem[b]).wait()
            pltpu.make_async_copy(dst_hbm.at[wid, ci], di[b], dsem[b]).wait()
            pltpu.sync_copy(rb[b], acc.at[di[b]], add=True)

        for b in range(NBUF):
            issue_idx(b, b)
        for b in range(NBUF - 1):
            issue_gather(b, b)

        M = NCHUNK // NBUF

        def body(i, carry):
            for b in range(NBUF):
                ci = i * NBUF + b
                issue_gather(ci + NBUF - 1, (b + NBUF - 1) % NBUF)
                drain_scatter(ci, b)
                issue_idx(ci + NBUF, b)
            return carry

        lax.fori_loop(0, M - 1, body, 0)
        for b in range(NBUF):
            ci = (M - 1) * NBUF + b
            if ci + NBUF - 1 < NCHUNK:
                issue_gather(ci + NBUF - 1, (b + NBUF - 1) % NBUF)
            drain_scatter(ci, b)
        plsc.subcore_barrier()

        pltpu.sync_copy(
            acc.at[pl.ds(row0, TILE_ROWS)],
            out_hbm.at[cid, pl.ds(row0, TILE_ROWS)],
        )

    return scatter_kernel(src3, dst3, h)


def _lin_relu(x, W, b):
    def body(x_ref, w_ref, b_ref, o_ref):
        acc = jnp.dot(x_ref[...], w_ref[...], preferred_element_type=jnp.float32)
        o_ref[...] = jnp.maximum(acc + b_ref[...], 0.0)

    return pl.pallas_call(
        body,
        grid=(NS,),
        in_specs=[
            pl.BlockSpec((TILE_ROWS, HID), lambda i: (i, 0)),
            pl.BlockSpec((HID, HID), lambda i: (0, 0)),
            pl.BlockSpec((1, HID), lambda i: (0, 0)),
        ],
        out_specs=pl.BlockSpec((TILE_ROWS, HID), lambda i: (i, 0)),
        out_shape=jax.ShapeDtypeStruct((N_PAD, HID), jnp.float32),
    )(x, W, b.reshape(1, HID))


def _layer_update(p0, p1, x0, Wc, beta):
    def body(p0_ref, p1_ref, x0_ref, w_ref, o_ref):
        xx = (p0_ref[...] + p1_ref[...]) * (1.0 - ALPHA_K) + ALPHA_K * x0_ref[...]
        mm = jnp.dot(xx, w_ref[...], preferred_element_type=jnp.float32)
        o_ref[...] = jnp.maximum((1.0 - beta) * xx + beta * mm, 0.0)

    return pl.pallas_call(
        body,
        grid=(NS,),
        in_specs=[
            pl.BlockSpec((TILE_ROWS, HID), lambda i: (i, 0)),
            pl.BlockSpec((TILE_ROWS, HID), lambda i: (i, 0)),
            pl.BlockSpec((TILE_ROWS, HID), lambda i: (i, 0)),
            pl.BlockSpec((HID, HID), lambda i: (0, 0)),
        ],
        out_specs=pl.BlockSpec((TILE_ROWS, HID), lambda i: (i, 0)),
        out_shape=jax.ShapeDtypeStruct((N_PAD, HID), jnp.float32),
    )(p0, p1, x0, Wc)


def kernel(x, edge_index, W_lin, b_lin, W1_1, W1_2):
    src = edge_index[0].astype(jnp.int32)
    dst = edge_index[1].astype(jnp.int32)
    pad = E_PAD - N_EDGES_K
    src_p = jnp.concatenate([src, jnp.zeros((pad,), jnp.int32)])
    dst_p = jnp.concatenate([dst, jnp.full((pad,), N_NODES_K, jnp.int32)])
    src_p = src_p.reshape(NW, NCHUNK, CHUNK)
    dst_p = dst_p.reshape(NW, NCHUNK, CHUNK)
    x_pad = jnp.pad(x, ((0, N_PAD - N_NODES_K), (0, 0)))

    h = _lin_relu(x_pad, W_lin, b_lin)
    x_cur = h
    for layer, Wc in enumerate((W1_1, W1_2)):
        beta = math.log(THETA_K / (layer + 1) + 1.0)
        p = _sc_segment_sum(src_p, dst_p, x_cur)
        x_cur = _layer_update(p[0], p[1], h, Wc, beta)
    return x_cur[:N_NODES_K]
